# baseline (speedup 1.0000x reference)
import functools

import jax
import jax.numpy as jnp
from jax.experimental import pallas as pl
from jax.experimental.pallas import tpu as pltpu

_KS = (0.8, 0.6)
_BM = 128


def _pad_to(x, m):
    return ((x + m - 1) // m) * m


def _blk(n):
    for b in (512, 256, 128):
        if n % b == 0:
            return b
    return _BM


def _gin_kern(a_ref, x_ref, xi_ref, w1_ref, b1_ref, w2_ref, b2_ref,
              fww_ref, out_ref, fw_ref):
    agg = jnp.dot(a_ref[...], x_ref[...],
                  preferred_element_type=jnp.float32)
    out = agg + xi_ref[...]
    h1 = jnp.maximum(
        jnp.dot(out, w1_ref[...], preferred_element_type=jnp.float32)
        + b1_ref[...], 0.0)
    h2 = jnp.dot(h1, w2_ref[...], preferred_element_type=jnp.float32) \
        + b2_ref[...]
    h2 = jnp.maximum(h2, 0.0)
    out_ref[...] = h2
    fw_ref[...] = jnp.dot(h2, fww_ref[...],
                          preferred_element_type=jnp.float32)


def _gin(A, x, p, fW):
    n = A.shape[0]
    dim = x.shape[1]
    bm = _blk(n)
    grid = (n // bm,)
    out, fw = pl.pallas_call(
        _gin_kern,
        grid=grid,
        in_specs=[
            pl.BlockSpec((bm, n), lambda i: (i, 0)),
            pl.BlockSpec((n, dim), lambda i: (0, 0)),
            pl.BlockSpec((bm, dim), lambda i: (i, 0)),
            pl.BlockSpec((dim, dim), lambda i: (0, 0)),
            pl.BlockSpec((1, dim), lambda i: (0, 0)),
            pl.BlockSpec((dim, dim), lambda i: (0, 0)),
            pl.BlockSpec((1, dim), lambda i: (0, 0)),
            pl.BlockSpec((dim, 1), lambda i: (0, 0)),
        ],
        out_specs=[
            pl.BlockSpec((bm, dim), lambda i: (i, 0)),
            pl.BlockSpec((bm, 1), lambda i: (i, 0)),
        ],
        out_shape=[
            jax.ShapeDtypeStruct((n, dim), jnp.float32),
            jax.ShapeDtypeStruct((n, 1), jnp.float32),
        ],
    )(A, x, x, p["W1"], p["b1"].reshape(1, dim), p["W2"],
      p["b2"].reshape(1, dim), fW)
    return out, fw


def _struct_kern(a_row, a_all, deg_ref, tri_ref, t2_ref):
    blk = a_row[...]
    prod = jnp.dot(blk, a_all[...], preferred_element_type=jnp.float32)
    tri_ref[...] = jnp.sum(prod * blk.astype(jnp.float32),
                           axis=1, keepdims=True)
    t2_ref[...] = jnp.dot(blk.astype(jnp.float32), deg_ref[...],
                          preferred_element_type=jnp.float32)


def _struct(Ab, deg):
    n = Ab.shape[0]
    bm = _blk(n)
    grid = (n // bm,)
    tri, t2 = pl.pallas_call(
        _struct_kern,
        grid=grid,
        in_specs=[
            pl.BlockSpec((bm, n), lambda i: (i, 0)),
            pl.BlockSpec((n, n), lambda i: (0, 0)),
            pl.BlockSpec((n, 1), lambda i: (0, 0)),
        ],
        out_specs=[
            pl.BlockSpec((bm, 1), lambda i: (i, 0)),
            pl.BlockSpec((bm, 1), lambda i: (i, 0)),
        ],
        out_shape=[
            jax.ShapeDtypeStruct((n, 1), jnp.float32),
            jax.ShapeDtypeStruct((n, 1), jnp.float32),
        ],
    )(Ab, Ab, deg)
    return tri, t2


def _ung_kern(hi_r, a_all, agb_r, idx_ref, ung_ref):
    aa = jnp.dot(hi_r[...], a_all[...], preferred_element_type=jnp.float32)
    bm, n = aa.shape
    cols = jax.lax.broadcasted_iota(jnp.int32, (bm, n), 1)
    notdiag = cols != idx_ref[...]
    ind = ((aa > 0.2) & notdiag).astype(jnp.bfloat16)
    ung_ref[...] = jnp.maximum(agb_r[...], ind)


def _ung(Ag_hi, Ab, Agb, idx_pad2d):
    kkp, n = Ag_hi.shape
    bm = _blk(kkp)
    grid = (kkp // bm,)
    return pl.pallas_call(
        _ung_kern,
        grid=grid,
        in_specs=[
            pl.BlockSpec((bm, n), lambda r: (r, 0)),
            pl.BlockSpec((n, n), lambda r: (0, 0)),
            pl.BlockSpec((bm, n), lambda r: (r, 0)),
            pl.BlockSpec((bm, 1), lambda r: (r, 0)),
        ],
        out_specs=pl.BlockSpec((bm, n), lambda r: (r, 0)),
        out_shape=jax.ShapeDtypeStruct((kkp, n), jnp.bfloat16),
    )(Ag_hi, Ab, Agb, idx_pad2d)


def _close_kern(u_row, u_all, af_ref, ab_ref, deg_ref, *, kk_true):
    r = pl.program_id(0)
    acc = jax.lax.dot_general(
        u_row[...], u_all[...], (((1,), (1,)), ((), ())),
        preferred_element_type=jnp.float32)
    bm, kkp = acc.shape
    rows = jax.lax.broadcasted_iota(jnp.int32, (bm, kkp), 0) + r * bm
    cols = jax.lax.broadcasted_iota(jnp.int32, (bm, kkp), 1)
    valid = (acc > 0.0) & (rows < kk_true) & (cols < kk_true)
    af = valid.astype(jnp.float32)
    af_ref[...] = af
    ab_ref[...] = af.astype(jnp.bfloat16)
    deg_ref[...] = jnp.sum(af, axis=1, keepdims=True)


def _close(U, kk_true):
    kkp, n = U.shape
    bm = _blk(kkp)
    grid = (kkp // bm,)
    return pl.pallas_call(
        functools.partial(_close_kern, kk_true=kk_true),
        grid=grid,
        in_specs=[
            pl.BlockSpec((bm, n), lambda r: (r, 0)),
            pl.BlockSpec((kkp, n), lambda r: (0, 0)),
        ],
        out_specs=[
            pl.BlockSpec((bm, kkp), lambda r: (r, 0)),
            pl.BlockSpec((bm, kkp), lambda r: (r, 0)),
            pl.BlockSpec((bm, 1), lambda r: (r, 0)),
        ],
        out_shape=[
            jax.ShapeDtypeStruct((kkp, kkp), jnp.float32),
            jax.ShapeDtypeStruct((kkp, kkp), jnp.bfloat16),
            jax.ShapeDtypeStruct((kkp, 1), jnp.float32),
        ],
    )(U, U)


def _pool_level(Ab, deg, n_true, d, fw, p, kfrac):
    kk = max(2, int(kfrac * n_true))
    kkp = _pad_to(kk, 512)

    tri, t2 = _struct(Ab, deg)
    t3 = jnp.dot(Ab, t2, preferred_element_type=jnp.float32)
    C = jnp.concatenate(
        [deg / (n_true - 1), deg, deg, t2, tri / 6.0, t3], axis=1)
    sw = (C @ p["sW"] + p["sb"])[:, 0]
    alpha = jnp.dot(sw[:n_true], p["aW"]) + p["ab"][0]
    scores = jax.nn.sigmoid(fw[:n_true, 0] + p["fb"][0] + alpha)
    values, idx = jax.lax.top_k(scores, kk)

    invlog = jnp.where(deg > 1.0, 1.0 / jnp.log(jnp.maximum(deg, 2.0)), 0.0)
    hi = invlog.astype(jnp.bfloat16)

    idx_pad = jnp.concatenate(
        [idx, jnp.zeros((kkp - kk,), jnp.int32)]).astype(jnp.int32)
    Agb = Ab[idx_pad]
    Ag_hi = Agb * hi[:, 0][None, :]

    U = _ung(Ag_hi, Ab, Agb, idx_pad[:, None])
    Af_n, Ab_n, deg_n = _close(U, kk)

    new_h = d[idx] * values[:, None]
    new_h = jnp.concatenate(
        [new_h, jnp.zeros((kkp - kk, d.shape[1]), jnp.float32)], axis=0)
    return Af_n, Ab_n, deg_n, new_h, idx, kk


def kernel(g, h, params):
    n0 = g.shape[0]
    dim = h.shape[1]
    zfW = jnp.zeros((dim, 1), jnp.float32)

    A0f = g
    A0b = g.astype(jnp.bfloat16)
    deg0 = jnp.sum(A0b, axis=1, dtype=jnp.float32, keepdims=True)

    d0, fw0 = _gin(A0f, h, params["down0"], params["pool0"]["fW"])
    A1f, A1b, deg1, h1, idx0, kk0 = _pool_level(
        A0b, deg0, n0, d0, fw0, params["pool0"], _KS[0])

    d1, fw1 = _gin(A1f, h1, params["down1"], params["pool1"]["fW"])
    A2f, A2b, deg2, h2, idx1, kk1 = _pool_level(
        A1b, deg1, kk0, d1, fw1, params["pool1"], _KS[1])

    hb, _ = _gin(A2f, h2, params["bottom"], zfW)

    n1p = A1f.shape[0]
    nh1 = jnp.zeros((n1p, dim), jnp.float32).at[idx1].set(hb[:kk1])
    u0, _ = _gin(A1f, nh1, params["up0"], zfW)
    u0 = u0 + d1

    nh0 = jnp.zeros((n0, dim), jnp.float32).at[idx0].set(u0[:kk0])
    u1, _ = _gin(A0f, nh0, params["up1"], zfW)
    u1 = u1 + d0

    return (u0[:kk0], u1, u1 + h)


# speedup vs baseline: 1.0817x; 1.0817x over previous
import functools

import jax
import jax.numpy as jnp
from jax.experimental import pallas as pl
from jax.experimental.pallas import tpu as pltpu

_KS = (0.8, 0.6)
_BM = 128


def _pad_to(x, m):
    return ((x + m - 1) // m) * m


def _blk(n):
    for b in (512, 256, 128):
        if n % b == 0:
            return b
    return _BM


def _gin_kern(a_ref, x_ref, xi_ref, w1_ref, b1_ref, w2_ref, b2_ref,
              fww_ref, out_ref, fw_ref):
    agg = jnp.dot(a_ref[...], x_ref[...],
                  preferred_element_type=jnp.float32)
    out = agg + xi_ref[...]
    h1 = jnp.maximum(
        jnp.dot(out, w1_ref[...], preferred_element_type=jnp.float32)
        + b1_ref[...], 0.0)
    h2 = jnp.dot(h1, w2_ref[...], preferred_element_type=jnp.float32) \
        + b2_ref[...]
    h2 = jnp.maximum(h2, 0.0)
    out_ref[...] = h2
    fw_ref[...] = jnp.dot(h2, fww_ref[...],
                          preferred_element_type=jnp.float32)


def _gin(A, x, p, fW):
    n = A.shape[0]
    dim = x.shape[1]
    bm = _blk(n)
    grid = (n // bm,)
    out, fw = pl.pallas_call(
        _gin_kern,
        grid=grid,
        in_specs=[
            pl.BlockSpec((bm, n), lambda i: (i, 0)),
            pl.BlockSpec((n, dim), lambda i: (0, 0)),
            pl.BlockSpec((bm, dim), lambda i: (i, 0)),
            pl.BlockSpec((dim, dim), lambda i: (0, 0)),
            pl.BlockSpec((1, dim), lambda i: (0, 0)),
            pl.BlockSpec((dim, dim), lambda i: (0, 0)),
            pl.BlockSpec((1, dim), lambda i: (0, 0)),
            pl.BlockSpec((dim, 1), lambda i: (0, 0)),
        ],
        out_specs=[
            pl.BlockSpec((bm, dim), lambda i: (i, 0)),
            pl.BlockSpec((bm, 1), lambda i: (i, 0)),
        ],
        out_shape=[
            jax.ShapeDtypeStruct((n, dim), jnp.float32),
            jax.ShapeDtypeStruct((n, 1), jnp.float32),
        ],
    )(A, x, x, p["W1"], p["b1"].reshape(1, dim), p["W2"],
      p["b2"].reshape(1, dim), fW)
    return out, fw


def _struct_kern(a_row, a_all, deg_ref, tri_ref, t2_ref):
    blk = a_row[...]
    prod = jnp.dot(blk, a_all[...], preferred_element_type=jnp.float32)
    tri_ref[...] = jnp.sum(prod * blk.astype(jnp.float32),
                           axis=1, keepdims=True)
    t2_ref[...] = jnp.dot(blk.astype(jnp.float32), deg_ref[...],
                          preferred_element_type=jnp.float32)


def _struct(Ab, deg):
    n = Ab.shape[0]
    bm = _blk(n)
    grid = (n // bm,)
    tri, t2 = pl.pallas_call(
        _struct_kern,
        grid=grid,
        in_specs=[
            pl.BlockSpec((bm, n), lambda i: (i, 0)),
            pl.BlockSpec((n, n), lambda i: (0, 0)),
            pl.BlockSpec((n, 1), lambda i: (0, 0)),
        ],
        out_specs=[
            pl.BlockSpec((bm, 1), lambda i: (i, 0)),
            pl.BlockSpec((bm, 1), lambda i: (i, 0)),
        ],
        out_shape=[
            jax.ShapeDtypeStruct((n, 1), jnp.float32),
            jax.ShapeDtypeStruct((n, 1), jnp.float32),
        ],
    )(Ab, Ab, deg)
    return tri, t2


def _ung_kern(hi_r, a_all, agb_r, idx_ref, ung_ref):
    aa = jnp.dot(hi_r[...], a_all[...], preferred_element_type=jnp.float32)
    bm, n = aa.shape
    cols = jax.lax.broadcasted_iota(jnp.int32, (bm, n), 1)
    notdiag = cols != idx_ref[...]
    ind = ((aa > 0.2) & notdiag).astype(jnp.bfloat16)
    ung_ref[...] = jnp.maximum(agb_r[...], ind)


def _ung(Ag_hi, Ab, Agb, idx_pad2d):
    kkp, n = Ag_hi.shape
    bm = _blk(kkp)
    grid = (kkp // bm,)
    return pl.pallas_call(
        _ung_kern,
        grid=grid,
        in_specs=[
            pl.BlockSpec((bm, n), lambda r: (r, 0)),
            pl.BlockSpec((n, n), lambda r: (0, 0)),
            pl.BlockSpec((bm, n), lambda r: (r, 0)),
            pl.BlockSpec((bm, 1), lambda r: (r, 0)),
        ],
        out_specs=pl.BlockSpec((bm, n), lambda r: (r, 0)),
        out_shape=jax.ShapeDtypeStruct((kkp, n), jnp.bfloat16),
    )(Ag_hi, Ab, Agb, idx_pad2d)


def _close_kern(u_row, u_all, af_ref, ab_ref, deg_ref, *, kk_true):
    r = pl.program_id(0)
    acc = jax.lax.dot_general(
        u_row[...], u_all[...], (((1,), (1,)), ((), ())),
        preferred_element_type=jnp.float32)
    bm, kkp = acc.shape
    rows = jax.lax.broadcasted_iota(jnp.int32, (bm, kkp), 0) + r * bm
    cols = jax.lax.broadcasted_iota(jnp.int32, (bm, kkp), 1)
    valid = (acc > 0.0) & (rows < kk_true) & (cols < kk_true)
    af = valid.astype(jnp.float32)
    af_ref[...] = af
    ab_ref[...] = af.astype(jnp.bfloat16)
    deg_ref[...] = jnp.sum(af, axis=1, keepdims=True)


def _close(U, kk_true):
    kkp, n = U.shape
    bm = _blk(kkp)
    grid = (kkp // bm,)
    return pl.pallas_call(
        functools.partial(_close_kern, kk_true=kk_true),
        grid=grid,
        in_specs=[
            pl.BlockSpec((bm, n), lambda r: (r, 0)),
            pl.BlockSpec((kkp, n), lambda r: (0, 0)),
        ],
        out_specs=[
            pl.BlockSpec((bm, kkp), lambda r: (r, 0)),
            pl.BlockSpec((bm, kkp), lambda r: (r, 0)),
            pl.BlockSpec((bm, 1), lambda r: (r, 0)),
        ],
        out_shape=[
            jax.ShapeDtypeStruct((kkp, kkp), jnp.float32),
            jax.ShapeDtypeStruct((kkp, kkp), jnp.bfloat16),
            jax.ShapeDtypeStruct((kkp, 1), jnp.float32),
        ],
    )(U, U)


def _pool_level(Ab, deg, n_true, d, fw, p, kfrac):
    kk = max(2, int(kfrac * n_true))
    kkp = _pad_to(kk, 256)

    tri, t2 = _struct(Ab, deg)
    t3 = jnp.dot(Ab, t2, preferred_element_type=jnp.float32)
    C = jnp.concatenate(
        [deg / (n_true - 1), deg, deg, t2, tri / 6.0, t3], axis=1)
    sw = (C @ p["sW"] + p["sb"])[:, 0]
    alpha = jnp.dot(sw[:n_true], p["aW"]) + p["ab"][0]
    scores = jax.nn.sigmoid(fw[:n_true, 0] + p["fb"][0] + alpha)
    values, idx = jax.lax.top_k(scores, kk)

    invlog = jnp.where(deg > 1.0, 1.0 / jnp.log(jnp.maximum(deg, 2.0)), 0.0)
    hi = invlog.astype(jnp.bfloat16)

    idx_pad = jnp.concatenate(
        [idx, jnp.zeros((kkp - kk,), jnp.int32)]).astype(jnp.int32)
    Agb = Ab[idx_pad]
    Ag_hi = Agb * hi[:, 0][None, :]

    U = _ung(Ag_hi, Ab, Agb, idx_pad[:, None])
    Af_n, Ab_n, deg_n = _close(U, kk)

    new_h = d[idx] * values[:, None]
    new_h = jnp.concatenate(
        [new_h, jnp.zeros((kkp - kk, d.shape[1]), jnp.float32)], axis=0)
    return Af_n, Ab_n, deg_n, new_h, idx, kk


def kernel(g, h, params):
    n0 = g.shape[0]
    dim = h.shape[1]
    zfW = jnp.zeros((dim, 1), jnp.float32)

    A0f = g
    A0b = g.astype(jnp.bfloat16)
    deg0 = jnp.sum(A0b, axis=1, dtype=jnp.float32, keepdims=True)

    d0, fw0 = _gin(A0f, h, params["down0"], params["pool0"]["fW"])
    A1f, A1b, deg1, h1, idx0, kk0 = _pool_level(
        A0b, deg0, n0, d0, fw0, params["pool0"], _KS[0])

    d1, fw1 = _gin(A1f, h1, params["down1"], params["pool1"]["fW"])
    A2f, A2b, deg2, h2, idx1, kk1 = _pool_level(
        A1b, deg1, kk0, d1, fw1, params["pool1"], _KS[1])

    hb, _ = _gin(A2f, h2, params["bottom"], zfW)

    n1p = A1f.shape[0]
    nh1 = jnp.zeros((n1p, dim), jnp.float32).at[idx1].set(hb[:kk1])
    u0, _ = _gin(A1f, nh1, params["up0"], zfW)
    u0 = u0 + d1

    nh0 = jnp.zeros((n0, dim), jnp.float32).at[idx0].set(u0[:kk0])
    u1, _ = _gin(A0f, nh0, params["up1"], zfW)
    u1 = u1 + d0

    return (u0[:kk0], u1, u1 + h)
